# baseline (speedup 1.0000x reference)
import functools

import jax
import jax.numpy as jnp
from jax import lax
from jax.experimental import pallas as pl
from jax.experimental.pallas import tpu as pltpu
from jax.experimental.pallas import tpu_sc as plsc

N_ATOMS = 100000
N_MOL = 3125

NC = 2
NS = 16
NW = NC * NS
L = 16

APW = 3200
N_PAD = NW * APW
CH = 128
NCH = APW // CH
M_PAD = 3200

LAST_REAL = N_ATOMS - (NW - 1) * APW
VPW = APW // L
UNROLL = 4
SCATTER_BATCH = 5

W_ENERGY = 1.0
W_FORCE = 10.0

_DNUMS = lax.GatherDimensionNumbers(
    offset_dims=(), collapsed_slice_dims=(0,), start_index_map=(0,))


def _dg(v, p):
    return lax.gather(v, p[:, None], _DNUMS, (1,),
                      mode=lax.GatherScatterMode.PROMISE_IN_BOUNDS)


def _sc_body(f_hbm, t_hbm, idx_hbm, ep_hbm, et_hbm,
             part_hbm, e2_hbm,
             f_v, t_v, idx_v, sq_v, cn_v,
             ep_v, et_v, e2_v, z_v,
             acc_sq, acc_cn, sem):
    c = lax.axis_index("c")
    s = lax.axis_index("s")
    wid = c * NS + s
    base = wid * APW

    dc = pltpu.async_copy(idx_hbm.at[wid], idx_v, sem)

    @pl.when(wid < NW - 1)
    def _load_full():
        da = pltpu.async_copy(f_hbm.at[pl.ds(base * 3, APW * 3)], f_v, sem)
        db = pltpu.async_copy(t_hbm.at[pl.ds(base * 3, APW * 3)], t_v, sem)
        da.wait()
        db.wait()

    @pl.when(wid == NW - 1)
    def _load_tail():
        n3 = LAST_REAL * 3
        da = pltpu.async_copy(f_hbm.at[pl.ds(base * 3, n3)],
                              f_v.at[pl.ds(0, n3)], sem)
        db = pltpu.async_copy(t_hbm.at[pl.ds(base * 3, n3)],
                              t_v.at[pl.ds(0, n3)], sem)
        da.wait()
        db.wait()

    @pl.when(s == 0)
    def _zero():
        zf = jnp.zeros((L,), jnp.float32)

        def zbody(k, _):
            z_v[pl.ds(k * L, L)] = zf
            return 0
        lax.fori_loop(0, M_PAD // L, zbody, 0)
        pltpu.sync_copy(z_v, acc_sq)
        pltpu.sync_copy(z_v, acc_cn)

    @pl.when(jnp.logical_and(c == 0, s == 1))
    def _energy():
        ea = pltpu.async_copy(ep_hbm, ep_v.at[pl.ds(0, N_MOL)], sem)
        eb = pltpu.async_copy(et_hbm, et_v.at[pl.ds(0, N_MOL)], sem)
        ea.wait()
        eb.wait()

        def ebody(k, _):
            sl = pl.ds(k * L, L)
            dd = ep_v[sl] - et_v[sl]
            e2_v[sl] = dd * dd
            return 0
        lax.fori_loop(0, M_PAD // L, ebody, 0)
        pltpu.sync_copy(e2_v, e2_hbm)

    dc.wait()

    a = lax.iota(jnp.int32, L)
    lo = jnp.zeros((L,), jnp.int32)
    hi = jnp.full((L,), L - 1, jnp.int32)

    def clamp(p):
        return jnp.minimum(jnp.maximum(p, lo), hi)

    px = (clamp(3 * a), clamp(3 * a - 16), clamp(3 * a - 32), a < 6, a < 11)
    py = (clamp(3 * a + 1), clamp(3 * a - 15), clamp(3 * a - 31), a < 5, a < 11)
    pz = (clamp(3 * a + 2), clamp(3 * a - 14), clamp(3 * a - 30), a < 5, a < 10)

    one = jnp.ones((L,), jnp.float32)
    zero = jnp.zeros((L,), jnp.float32)

    def comp(v0, v1, v2, p):
        p0, p1, p2, m0, m1 = p
        return jnp.where(m0, _dg(v0, p0),
                         jnp.where(m1, _dg(v1, p1), _dg(v2, p2)))

    def body(jo, _):
        for ju in range(UNROLL):
            j = jo * UNROLL + ju
            o = j * (3 * L)
            d0 = f_v[pl.ds(o, L)] - t_v[pl.ds(o, L)]
            d1 = f_v[pl.ds(o + L, L)] - t_v[pl.ds(o + L, L)]
            d2 = f_v[pl.ds(o + 2 * L, L)] - t_v[pl.ds(o + 2 * L, L)]
            v0 = d0 * d0
            v1 = d1 * d1
            v2 = d2 * d2
            g = base + j * L + a
            valid = g < N_ATOMS
            sq = comp(v0, v1, v2, px) + comp(v0, v1, v2, py) + \
                comp(v0, v1, v2, pz)
            sq = jnp.where(valid, sq, zero)
            cn = jnp.where(valid, one, zero)
            sq_v[j // (CH // L), pl.ds((j % (CH // L)) * L, L)] = sq
            cn_v[j // (CH // L), pl.ds((j % (CH // L)) * L, L)] = cn
        return 0

    lax.fori_loop(0, VPW // UNROLL, body, 0)

    plsc.subcore_barrier()

    for g0 in range(0, NCH, SCATTER_BATCH):
        descs = []
        for ch in range(g0, min(g0 + SCATTER_BATCH, NCH)):
            descs.append(pltpu.async_copy(
                sq_v.at[ch], acc_sq.at[idx_v.at[ch]], sem, add=True))
            descs.append(pltpu.async_copy(
                cn_v.at[ch], acc_cn.at[idx_v.at[ch]], sem, add=True))
        for dsc in descs:
            dsc.wait()

    plsc.subcore_barrier()

    @pl.when(s == 0)
    def _writeback():
        wa = pltpu.async_copy(acc_sq, part_hbm.at[c, 0], sem)
        wb = pltpu.async_copy(acc_cn, part_hbm.at[c, 1], sem)
        wa.wait()
        wb.wait()


_sc_loss = functools.partial(
    pl.kernel,
    out_type=(
        jax.ShapeDtypeStruct((NC, 2, M_PAD), jnp.float32),
        jax.ShapeDtypeStruct((M_PAD,), jnp.float32),
    ),
    mesh=plsc.VectorSubcoreMesh(core_axis_name="c", subcore_axis_name="s"),
    scratch_types=[
        pltpu.VMEM((APW * 3,), jnp.float32),
        pltpu.VMEM((APW * 3,), jnp.float32),
        pltpu.VMEM((NCH, CH), jnp.int32),
        pltpu.VMEM((NCH, CH), jnp.float32),
        pltpu.VMEM((NCH, CH), jnp.float32),
        pltpu.VMEM((M_PAD,), jnp.float32),
        pltpu.VMEM((M_PAD,), jnp.float32),
        pltpu.VMEM((M_PAD,), jnp.float32),
        pltpu.VMEM((M_PAD,), jnp.float32),
        pltpu.VMEM_SHARED((M_PAD,), jnp.float32),
        pltpu.VMEM_SHARED((M_PAD,), jnp.float32),
        pltpu.SemaphoreType.DMA,
    ],
)(_sc_body)


def kernel(force_pred, force_true, energy_pred, energy_true, atom_mol_idx,
           num_molecules):
    f_flat = force_pred.reshape(-1)
    t_flat = force_true.reshape(-1)
    idx3 = jnp.pad(atom_mol_idx, (0, N_PAD - N_ATOMS)).reshape(NW, NCH, CH)

    part, e2 = _sc_loss(f_flat, t_flat, idx3, energy_pred, energy_true)

    sq = part[0, 0, :N_MOL] + part[1, 0, :N_MOL]
    cnt = jnp.maximum(part[0, 1, :N_MOL] + part[1, 1, :N_MOL], 1.0)
    force_loss = sq / cnt
    energy_loss = jnp.mean(e2[:N_MOL] / cnt)
    return W_ENERGY * energy_loss + W_FORCE * force_loss


# speedup vs baseline: 4.4825x; 4.4825x over previous
import functools

import jax
import jax.numpy as jnp
from jax import lax
from jax.experimental import pallas as pl
from jax.experimental.pallas import tpu as pltpu
from jax.experimental.pallas import tpu_sc as plsc

N_ATOMS = 100000
N_MOL = 3125

NC = 2
NS = 16
NW = NC * NS
L = 16

APW = 3200
N_PAD = NW * APW
CH = 128
NCH = APW // CH
M_PAD = 3200

VPW = APW // L
UNROLL = 4
SCATTER_BATCH = 5

W_ENERGY = 1.0
W_FORCE = 10.0


def _sc_body(d_hbm, idx_hbm, ep_hbm, et_hbm,
             part_hbm, e2_hbm,
             d_v, idx_v, sq_v, cn_v,
             ep_v, et_v, e2_v, z_v,
             acc_sq, acc_cn, sem):
    c = lax.axis_index("c")
    s = lax.axis_index("s")
    wid = c * NS + s
    base = wid * APW

    da = pltpu.async_copy(d_hbm.at[pl.ds(base, APW)],
                          d_v.at[pl.ds(0, APW)], sem)
    db = pltpu.async_copy(d_hbm.at[pl.ds(N_PAD + base, APW)],
                          d_v.at[pl.ds(APW, APW)], sem)
    de = pltpu.async_copy(d_hbm.at[pl.ds(2 * N_PAD + base, APW)],
                          d_v.at[pl.ds(2 * APW, APW)], sem)
    dc = pltpu.async_copy(idx_hbm.at[wid], idx_v, sem)

    @pl.when(s == 0)
    def _zero():
        zf = jnp.zeros((L,), jnp.float32)

        def zbody(k, _):
            z_v[pl.ds(k * L, L)] = zf
            return 0
        lax.fori_loop(0, M_PAD // L, zbody, 0)
        pltpu.sync_copy(z_v, acc_sq)
        pltpu.sync_copy(z_v, acc_cn)

    @pl.when(jnp.logical_and(c == 0, s == 1))
    def _energy():
        ea = pltpu.async_copy(ep_hbm, ep_v.at[pl.ds(0, N_MOL)], sem)
        eb = pltpu.async_copy(et_hbm, et_v.at[pl.ds(0, N_MOL)], sem)
        ea.wait()
        eb.wait()

        def ebody(k, _):
            sl = pl.ds(k * L, L)
            dd = ep_v[sl] - et_v[sl]
            e2_v[sl] = dd * dd
            return 0
        lax.fori_loop(0, M_PAD // L, ebody, 0)
        pltpu.sync_copy(e2_v, e2_hbm)

    da.wait()
    db.wait()
    de.wait()
    dc.wait()

    iota = lax.iota(jnp.int32, L)
    one = jnp.ones((L,), jnp.float32)
    zero = jnp.zeros((L,), jnp.float32)

    def body(jo, _):
        for ju in range(UNROLL):
            j = jo * UNROLL + ju
            dx = d_v[pl.ds(j * L, L)]
            dy = d_v[pl.ds(APW + j * L, L)]
            dz = d_v[pl.ds(2 * APW + j * L, L)]
            sq = dx * dx + dy * dy + dz * dz
            g = base + j * L + iota
            valid = g < N_ATOMS
            cn = jnp.where(valid, one, zero)
            sq_v[j // (CH // L), pl.ds((j % (CH // L)) * L, L)] = sq
            cn_v[j // (CH // L), pl.ds((j % (CH // L)) * L, L)] = cn
        return 0

    lax.fori_loop(0, VPW // UNROLL, body, 0)

    plsc.subcore_barrier()

    for g0 in range(0, NCH, SCATTER_BATCH):
        descs = []
        for ch in range(g0, min(g0 + SCATTER_BATCH, NCH)):
            descs.append(pltpu.async_copy(
                sq_v.at[ch], acc_sq.at[idx_v.at[ch]], sem, add=True))
            descs.append(pltpu.async_copy(
                cn_v.at[ch], acc_cn.at[idx_v.at[ch]], sem, add=True))
        for dsc in descs:
            dsc.wait()

    plsc.subcore_barrier()

    @pl.when(s == 0)
    def _writeback():
        wa = pltpu.async_copy(acc_sq, part_hbm.at[c, 0], sem)
        wb = pltpu.async_copy(acc_cn, part_hbm.at[c, 1], sem)
        wa.wait()
        wb.wait()


_sc_loss = functools.partial(
    pl.kernel,
    out_type=(
        jax.ShapeDtypeStruct((NC, 2, M_PAD), jnp.float32),
        jax.ShapeDtypeStruct((M_PAD,), jnp.float32),
    ),
    mesh=plsc.VectorSubcoreMesh(core_axis_name="c", subcore_axis_name="s"),
    scratch_types=[
        pltpu.VMEM((3 * APW,), jnp.float32),
        pltpu.VMEM((NCH, CH), jnp.int32),
        pltpu.VMEM((NCH, CH), jnp.float32),
        pltpu.VMEM((NCH, CH), jnp.float32),
        pltpu.VMEM((M_PAD,), jnp.float32),
        pltpu.VMEM((M_PAD,), jnp.float32),
        pltpu.VMEM((M_PAD,), jnp.float32),
        pltpu.VMEM((M_PAD,), jnp.float32),
        pltpu.VMEM_SHARED((M_PAD,), jnp.float32),
        pltpu.VMEM_SHARED((M_PAD,), jnp.float32),
        pltpu.SemaphoreType.DMA,
    ],
)(_sc_body)


def kernel(force_pred, force_true, energy_pred, energy_true, atom_mol_idx,
           num_molecules):
    pad = N_PAD - N_ATOMS
    dT = jnp.pad((force_pred - force_true).T, ((0, 0), (0, pad)))
    idx3 = jnp.pad(atom_mol_idx, (0, pad)).reshape(NW, NCH, CH)

    part, e2 = _sc_loss(dT.reshape(-1), idx3, energy_pred, energy_true)

    sq = part[0, 0, :N_MOL] + part[1, 0, :N_MOL]
    cnt = jnp.maximum(part[0, 1, :N_MOL] + part[1, 1, :N_MOL], 1.0)
    force_loss = sq / cnt
    energy_loss = jnp.mean(e2[:N_MOL] / cnt)
    return W_ENERGY * energy_loss + W_FORCE * force_loss
